# Initial kernel scaffold; baseline (speedup 1.0000x reference)
import jax
import jax.numpy as jnp
from jax.experimental import pallas as pl


def kernel(x, table):
    raise NotImplementedError("write your pallas kernel here")


# speedup vs baseline: 5.1659x; 5.1659x over previous
import functools

import jax
import jax.numpy as jnp
from jax import lax
from jax.experimental import pallas as pl
from jax.experimental.pallas import tpu as pltpu
from jax.experimental.pallas import tpu_sc as plsc

DIM = 128
NC = 2
NS = 16
NW = NC * NS
GCH = 128


@functools.partial(jax.jit, static_argnames=("total",))
def _embed_gather(idx, table, total):
    b_per_w = total // NW
    n_g = b_per_w // GCH
    mesh = plsc.VectorSubcoreMesh(core_axis_name="c", subcore_axis_name="s")

    @functools.partial(
        pl.kernel,
        mesh=mesh,
        out_type=jax.ShapeDtypeStruct((total, DIM), jnp.float32),
        scratch_types=[
            pltpu.VMEM((1, GCH), jnp.int32),
            pltpu.VMEM((1, GCH, DIM), jnp.float32),
            pltpu.SemaphoreType.DMA,
        ],
    )
    def k(idx_hbm, table_hbm, out_hbm, idx_v, rows_v, gsem):
        wid = lax.axis_index("s") * NC + lax.axis_index("c")
        base = wid * b_per_w

        def body(g, _):
            off = base + g * GCH
            pltpu.sync_copy(idx_hbm.at[pl.ds(off, GCH)], idx_v.at[0])
            pltpu.async_copy(table_hbm.at[idx_v.at[0]], rows_v.at[0], gsem).wait()
            pltpu.sync_copy(rows_v.at[0], out_hbm.at[pl.ds(off, GCH)])
            return ()

        lax.fori_loop(0, n_g, body, ())

    return k(idx, table)


def kernel(x, table):
    b, h = x.shape
    idx = x.reshape(-1).astype(jnp.int32)
    out = _embed_gather(idx, table, b * h)
    return out.reshape(b, h, DIM)


# speedup vs baseline: 9.1785x; 1.7767x over previous
import functools

import jax
import jax.numpy as jnp
from jax import lax
from jax.experimental import pallas as pl
from jax.experimental.pallas import tpu as pltpu
from jax.experimental.pallas import tpu_sc as plsc

DIM = 128
NC = 2
NS = 16
NW = NC * NS
GCH = 128
NBUF = 4
PF = 2


@functools.partial(jax.jit, static_argnames=("total",))
def _embed_gather(idx3, table, total):
    b_per_w = total // NW
    n_g = b_per_w // GCH
    mesh = plsc.VectorSubcoreMesh(core_axis_name="c", subcore_axis_name="s")

    @functools.partial(
        pl.kernel,
        mesh=mesh,
        out_type=jax.ShapeDtypeStruct((total, DIM), jnp.float32),
        scratch_types=[
            pltpu.VMEM((n_g, GCH), jnp.int32),
            pltpu.VMEM((NBUF, GCH, DIM), jnp.float32),
            pltpu.SemaphoreType.DMA((NBUF,)),
            pltpu.SemaphoreType.DMA((NBUF,)),
        ],
    )
    def k(idx_hbm, table_hbm, out_hbm, idx_v, rows_v, gsem, ssem):
        wid = lax.axis_index("s") * NC + lax.axis_index("c")
        base = wid * b_per_w

        pltpu.sync_copy(idx_hbm.at[wid], idx_v)

        def fire(g, b):
            pltpu.async_copy(table_hbm.at[idx_v.at[g]], rows_v.at[b], gsem.at[b])

        def wait_gather(g, b):
            pltpu.make_async_copy(
                table_hbm.at[idx_v.at[g]], rows_v.at[b], gsem.at[b]
            ).wait()

        def scatter(g, b):
            dst = out_hbm.at[pl.ds(base + g * GCH, GCH)]
            pltpu.async_copy(rows_v.at[b], dst, ssem.at[b])

        def wait_scatter(g, b):
            dst = out_hbm.at[pl.ds(base + g * GCH, GCH)]
            pltpu.make_async_copy(rows_v.at[b], dst, ssem.at[b]).wait()

        for g in range(PF):
            fire(g, g % NBUF)

        def outer(t, _):
            g0 = t * NBUF
            for j in range(NBUF):
                g = g0 + j
                wait_gather(g, j)
                scatter(g, j)
                gn = g + PF
                bn = (j + PF) % NBUF

                @pl.when(gn < n_g)
                def _():
                    @pl.when(gn >= NBUF)
                    def _():
                        wait_scatter(gn - NBUF, bn)

                    fire(gn, bn)

            return ()

        lax.fori_loop(0, n_g // NBUF, outer, ())

        for i in range(NBUF):
            g = n_g - NBUF + i
            wait_scatter(g, g % NBUF)

    return k(idx3, table)


def kernel(x, table):
    b, h = x.shape
    total = b * h
    idx3 = x.reshape(NW, total // NW // GCH, GCH).astype(jnp.int32)
    out = _embed_gather(idx3, table, total)
    return out.reshape(b, h, DIM)
